# Initial kernel scaffold; baseline (speedup 1.0000x reference)
import jax
import jax.numpy as jnp
from jax.experimental import pallas as pl


def kernel(x, task_id, task_emb, gate_W, gate_b, W1, b1, W2, b2):
    raise NotImplementedError("write your pallas kernel here")


# speedup vs baseline: 6.8812x; 6.8812x over previous
import jax
import jax.numpy as jnp
from jax import lax
from jax.experimental import pallas as pl

D_MODEL = 2048
NUM_EXPERTS = 8
TOP_K = 2
NUM_TASKS = 64
D_TASK_EMBED = 64
D_FFN = 32
EF = NUM_EXPERTS * D_FFN

TOK_BLK = 512


def _stage1_body(x_ref, w1_ref, b1_ref, temb_ref, gwt_ref, gb_ref,
                 h_ref, s1_ref, s2_ref, te_ref):
    x = x_ref[...]
    s1_ref[...] = jnp.full((1, 1, 128), jnp.sum(x), dtype=jnp.float32)
    s2_ref[...] = jnp.full((1, 1, 128), jnp.sum(x * x), dtype=jnp.float32)
    h = jnp.dot(x, w1_ref[...], preferred_element_type=jnp.float32)
    h = h + b1_ref[...]
    h_ref[...] = h * jax.nn.sigmoid(h)
    te_ref[...] = jnp.dot(temb_ref[...].astype(jnp.bfloat16),
                          gwt_ref[...].astype(jnp.bfloat16),
                          preferred_element_type=jnp.float32) + gb_ref[...]


def _stage2_body(x_ref, h_ref, mean_ref, std_ref, te_ref, gwx_ref,
                 w2_ref, b2_ref, exp_ref, out_ref, idx_ref, cnt_ref):
    i = pl.program_id(0)
    b = i // (pl.num_programs(0) // mean_ref.shape[0])
    mean_row = mean_ref[pl.ds(b, 1), :][:, :1]
    std_row = std_ref[pl.ds(b, 1), :][:, :1]
    te_row = te_ref[pl.ds(b, 1), :]

    xn = (x_ref[...] - mean_row) / std_row
    logits = jnp.dot(xn.astype(jnp.bfloat16),
                     gwx_ref[...].astype(jnp.bfloat16),
                     preferred_element_type=jnp.float32) + te_row

    ii = lax.broadcasted_iota(jnp.int32, logits.shape, 1)
    m1 = jnp.max(logits, axis=1, keepdims=True)
    i1 = jnp.min(jnp.where(logits == m1, ii, NUM_EXPERTS), axis=1, keepdims=True)
    l2 = jnp.where(ii == i1, -jnp.inf, logits)
    m2 = jnp.max(l2, axis=1, keepdims=True)
    i2 = jnp.min(jnp.where(l2 == m2, ii, NUM_EXPERTS), axis=1, keepdims=True)
    ed = jnp.exp(m2 - m1)
    p1 = 1.0 / (1.0 + ed)
    p2 = ed / (1.0 + ed)
    oh1 = (ii == i1)
    oh2 = (ii == i2)
    w = jnp.where(oh1, p1, 0.0) + jnp.where(oh2, p2, 0.0)

    idx_ref[...] = jnp.concatenate([i1, i2], axis=1)
    cnt = jnp.sum(oh1.astype(jnp.float32) + oh2.astype(jnp.float32),
                  axis=0, keepdims=True)
    cnt_ref[...] = cnt[None]

    w_exp = jnp.dot(w, exp_ref[...], preferred_element_type=jnp.float32)
    hw = h_ref[...] * w_exp
    out = jnp.dot(hw, w2_ref[...], preferred_element_type=jnp.float32)
    out = out + jnp.dot(w, b2_ref[...], preferred_element_type=jnp.float32)
    out_ref[...] = out


@jax.jit
def kernel(x, task_id, task_emb, gate_W, gate_b, W1, b1, W2, b2):
    B, S, D = x.shape
    T = B * S
    nblk = T // TOK_BLK
    blk_per_b = nblk // B

    x2d = x.reshape(T, D)
    w1a = W1.transpose(1, 0, 2).reshape(D, EF)
    b1f = b1.reshape(1, EF)
    gwx = gate_W[:D, :]
    gwt = gate_W[D:, :]
    gbr = gate_b.reshape(1, NUM_EXPERTS)
    w2a = W2.reshape(EF, D)

    grid1 = (nblk,)
    h, s1, s2, te_all = pl.pallas_call(
        _stage1_body,
        grid=grid1,
        in_specs=[
            pl.BlockSpec((TOK_BLK, D), lambda i: (i, 0)),
            pl.BlockSpec((D, EF), lambda i: (0, 0)),
            pl.BlockSpec((1, EF), lambda i: (0, 0)),
            pl.BlockSpec((NUM_TASKS, D_TASK_EMBED), lambda i: (0, 0)),
            pl.BlockSpec((D_TASK_EMBED, NUM_EXPERTS), lambda i: (0, 0)),
            pl.BlockSpec((1, NUM_EXPERTS), lambda i: (0, 0)),
        ],
        out_specs=[
            pl.BlockSpec((TOK_BLK, EF), lambda i: (i, 0)),
            pl.BlockSpec((1, 1, 128), lambda i: (i, 0, 0)),
            pl.BlockSpec((1, 1, 128), lambda i: (i, 0, 0)),
            pl.BlockSpec((NUM_TASKS, NUM_EXPERTS), lambda i: (0, 0)),
        ],
        out_shape=[
            jax.ShapeDtypeStruct((T, EF), jnp.float32),
            jax.ShapeDtypeStruct((nblk, 1, 128), jnp.float32),
            jax.ShapeDtypeStruct((nblk, 1, 128), jnp.float32),
            jax.ShapeDtypeStruct((NUM_TASKS, NUM_EXPERTS), jnp.float32),
        ],
    )(x2d, w1a, b1f, task_emb, gwt, gbr)

    n = jnp.float32(S * D)
    bsum = s1[:, 0, 0].reshape(B, blk_per_b).sum(axis=1)
    bsq = s2[:, 0, 0].reshape(B, blk_per_b).sum(axis=1)
    mean = bsum / n
    var = bsq / n - mean * mean
    std = jnp.sqrt(var + 1e-5)
    mean_mat = jnp.broadcast_to(mean[:, None], (B, NUM_EXPERTS))
    std_mat = jnp.broadcast_to(std[:, None], (B, NUM_EXPERTS))
    te_row = te_all[task_id]

    expand = jnp.repeat(jnp.eye(NUM_EXPERTS, dtype=jnp.float32), D_FFN, axis=1)
    expand = expand.reshape(NUM_EXPERTS, EF)

    out, idx, cnt = pl.pallas_call(
        _stage2_body,
        grid=grid1,
        in_specs=[
            pl.BlockSpec((TOK_BLK, D), lambda i: (i, 0)),
            pl.BlockSpec((TOK_BLK, EF), lambda i: (i, 0)),
            pl.BlockSpec((B, NUM_EXPERTS), lambda i: (0, 0)),
            pl.BlockSpec((B, NUM_EXPERTS), lambda i: (0, 0)),
            pl.BlockSpec((B, NUM_EXPERTS), lambda i: (0, 0)),
            pl.BlockSpec((D, NUM_EXPERTS), lambda i: (0, 0)),
            pl.BlockSpec((EF, D), lambda i: (0, 0)),
            pl.BlockSpec((NUM_EXPERTS, D), lambda i: (0, 0)),
            pl.BlockSpec((NUM_EXPERTS, EF), lambda i: (0, 0)),
        ],
        out_specs=[
            pl.BlockSpec((TOK_BLK, D), lambda i: (i, 0)),
            pl.BlockSpec((TOK_BLK, TOP_K), lambda i: (i, 0)),
            pl.BlockSpec((1, 1, NUM_EXPERTS), lambda i: (i, 0, 0)),
        ],
        out_shape=[
            jax.ShapeDtypeStruct((T, D), jnp.float32),
            jax.ShapeDtypeStruct((T, TOP_K), jnp.int32),
            jax.ShapeDtypeStruct((nblk, 1, NUM_EXPERTS), jnp.float32),
        ],
    )(x2d, h, mean_mat, std_mat, te_row, gwx, w2a, b2, expand)

    final_output = out.reshape(B, S, D)
    topk_idx = idx.reshape(B, S, TOP_K)

    counts = jnp.sum(cnt, axis=(0, 1))
    usage_mean = jnp.mean(counts) + 1e-6
    usage_std = jnp.std(counts, ddof=1)
    lb_loss = (usage_std / usage_mean) ** 2
    return (final_output, lb_loss, topk_idx)
